# Initial kernel scaffold; baseline (speedup 1.0000x reference)
import jax
import jax.numpy as jnp
from jax.experimental import pallas as pl


def kernel(x, edge_index, batch, W1, b1, W2, b2, Wg1, bg1, Wg2, bg2, Wo, bo):
    raise NotImplementedError("write your pallas kernel here")


# speedup vs baseline: 12.3606x; 12.3606x over previous
import jax
import jax.numpy as jnp
from jax import lax
from jax.experimental import pallas as pl
from jax.experimental.pallas import tpu as pltpu
from jax.experimental.pallas import tpu_sc as plsc

N = 10000
E = 320000
D = 128
NG = 64

NC = 2
NS = 16
L = 16
NW = NC * NS

CH = 128

BR = 512
GRID = 20
NPAD = GRID * BR
RPT = NPAD // NS

_mesh = plsc.VectorSubcoreMesh(
    core_axis_name="c", subcore_axis_name="s", num_cores=NC, num_subcores=NS)


def _zero_vmem_2d(ref, nrows, ncols):
    npc = ncols // L

    def body(i, _):
        ref[i // npc, pl.ds((i % npc) * L, L)] = jnp.zeros((L,), jnp.float32)
        return 0

    lax.fori_loop(0, nrows * npc, body, 0)


def _fill_spmem_zero(acc, zbuf, row0, nrows, zrows):
    nfull = nrows // zrows
    rem = nrows - nfull * zrows
    for j in range(nfull):
        pltpu.sync_copy(zbuf.at[pl.ds(0, zrows)],
                        acc.at[pl.ds(row0 + j * zrows, zrows)])
    if rem:
        pltpu.sync_copy(zbuf.at[pl.ds(0, rem)],
                        acc.at[pl.ds(row0 + nfull * zrows, rem)])


EPT1 = E // NW
NCH1 = EPT1 // CH
REM1 = EPT1 - NCH1 * CH


def _deg_body(dst_hbm, out_hbm, ones_v, ones_r, idx_v, idx_r, acc):
    c = lax.axis_index("c")
    s = lax.axis_index("s")
    wid = c * NS + s
    base = wid * EPT1
    row0 = s * RPT

    _zero_vmem_2d(ones_v, CH, L)
    _fill_spmem_zero(acc, ones_v, row0, RPT, CH)

    def fill1(i, _):
        ones_v[i] = jnp.ones((L,), jnp.float32)
        return 0
    lax.fori_loop(0, CH, fill1, 0)

    def fill1r(i, _):
        ones_r[i] = jnp.ones((L,), jnp.float32)
        return 0
    lax.fori_loop(0, REM1, fill1r, 0)

    plsc.subcore_barrier()

    def chunk(k, _):
        off = base + k * CH
        pltpu.sync_copy(dst_hbm.at[pl.ds(off, CH)], idx_v)
        pltpu.sync_copy(ones_v, acc.at[idx_v], add=True)
        return 0
    lax.fori_loop(0, NCH1, chunk, 0)
    if REM1:
        pltpu.sync_copy(dst_hbm.at[pl.ds(base + NCH1 * CH, REM1)], idx_r)
        pltpu.sync_copy(ones_r, acc.at[idx_r], add=True)

    plsc.subcore_barrier()
    pltpu.sync_copy(acc.at[pl.ds(row0, RPT)],
                    out_hbm.at[c, pl.ds(row0, RPT)])


_deg_kernel = pl.kernel(
    _deg_body,
    out_type=jax.ShapeDtypeStruct((NC, NPAD, L), jnp.float32),
    mesh=_mesh,
    scratch_types=[
        pltpu.VMEM((CH, L), jnp.float32),
        pltpu.VMEM((REM1, L), jnp.float32),
        pltpu.VMEM((CH,), jnp.int32),
        pltpu.VMEM((REM1,), jnp.int32),
        pltpu.VMEM_SHARED((NPAD, L), jnp.float32),
    ],
)


EPT2 = E // NS
NCH2 = EPT2 // CH
REM2 = EPT2 - NCH2 * CH


def _gather_scatter_range(src_hbm, dst_hbm, g_hbm, acc,
                          sbuf, dbuf, sbuf_r, dbuf_r, rows, rows_r, sem,
                          base, nch, rem):
    def chunk(k, _):
        off = base + k * CH
        pltpu.sync_copy(src_hbm.at[pl.ds(off, CH)], sbuf)
        pltpu.sync_copy(dst_hbm.at[pl.ds(off, CH)], dbuf)
        pltpu.async_copy(g_hbm.at[sbuf], rows, sem).wait()
        pltpu.sync_copy(rows, acc.at[dbuf], add=True)
        return 0
    lax.fori_loop(0, nch, chunk, 0)
    if rem:
        off = base + nch * CH
        pltpu.sync_copy(src_hbm.at[pl.ds(off, rem)], sbuf_r)
        pltpu.sync_copy(dst_hbm.at[pl.ds(off, rem)], dbuf_r)
        pltpu.async_copy(g_hbm.at[sbuf_r], rows_r, sem).wait()
        pltpu.sync_copy(rows_r, acc.at[dbuf_r], add=True)


def _agg1_body(src_hbm, dst_hbm, g_hbm, out_hbm,
               sbuf, dbuf, sbuf_r, dbuf_r, rows, rows_r, zbuf, sem, acc):
    c = lax.axis_index("c")
    s = lax.axis_index("s")
    wid = c * NS + s
    row0 = s * RPT

    _zero_vmem_2d(zbuf, CH, D)
    _fill_spmem_zero(acc, zbuf, row0, RPT, CH)
    plsc.subcore_barrier()

    _gather_scatter_range(src_hbm, dst_hbm, g_hbm, acc,
                          sbuf, dbuf, sbuf_r, dbuf_r, rows, rows_r, sem,
                          wid * EPT1, NCH1, REM1)

    plsc.subcore_barrier()
    pltpu.sync_copy(acc.at[pl.ds(row0, RPT)],
                    out_hbm.at[c, pl.ds(row0, RPT)])


_agg1_kernel = pl.kernel(
    _agg1_body,
    out_type=jax.ShapeDtypeStruct((NC, NPAD, D), jnp.float32),
    mesh=_mesh,
    scratch_types=[
        pltpu.VMEM((CH,), jnp.int32),
        pltpu.VMEM((CH,), jnp.int32),
        pltpu.VMEM((REM1,), jnp.int32),
        pltpu.VMEM((REM1,), jnp.int32),
        pltpu.VMEM((CH, D), jnp.float32),
        pltpu.VMEM((REM1, D), jnp.float32),
        pltpu.VMEM((CH, D), jnp.float32),
        pltpu.SemaphoreType.DMA,
        pltpu.VMEM_SHARED((NPAD, D), jnp.float32),
    ],
)


def _agg2_body(src_hbm, dst_hbm, ga_hbm, gb_hbm, out_hbm,
               sbuf, dbuf, sbuf_r, dbuf_r, rows, rows_r, zbuf, sem, acc):
    c = lax.axis_index("c")
    s = lax.axis_index("s")
    base = s * EPT2
    row0 = s * RPT

    _zero_vmem_2d(zbuf, CH, D)
    _fill_spmem_zero(acc, zbuf, row0, RPT, CH)
    plsc.subcore_barrier()

    @pl.when(c == 0)
    def _():
        _gather_scatter_range(src_hbm, dst_hbm, ga_hbm, acc,
                              sbuf, dbuf, sbuf_r, dbuf_r, rows, rows_r, sem,
                              base, NCH2, REM2)

    @pl.when(c == 1)
    def _():
        _gather_scatter_range(src_hbm, dst_hbm, gb_hbm, acc,
                              sbuf, dbuf, sbuf_r, dbuf_r, rows, rows_r, sem,
                              base, NCH2, REM2)

    plsc.subcore_barrier()
    pltpu.sync_copy(acc.at[pl.ds(row0, RPT)],
                    out_hbm.at[c, pl.ds(row0, RPT)])


_agg2_kernel = pl.kernel(
    _agg2_body,
    out_type=jax.ShapeDtypeStruct((NC, NPAD, D), jnp.float32),
    mesh=_mesh,
    scratch_types=[
        pltpu.VMEM((CH,), jnp.int32),
        pltpu.VMEM((CH,), jnp.int32),
        pltpu.VMEM((REM2,), jnp.int32),
        pltpu.VMEM((REM2,), jnp.int32),
        pltpu.VMEM((CH, D), jnp.float32),
        pltpu.VMEM((REM2, D), jnp.float32),
        pltpu.VMEM((CH, D), jnp.float32),
        pltpu.SemaphoreType.DMA,
        pltpu.VMEM_SHARED((NPAD, D), jnp.float32),
    ],
)


GCHUNK = 64
D2 = 256


def _segmax_body(batch_hbm, h_hbm, out_hbm, bbuf, cbuf, obuf):
    wid = lax.axis_index("c") * NS + lax.axis_index("s")
    g0 = wid * 2

    pltpu.sync_copy(batch_hbm, bbuf)

    t0 = lax.broadcast(g0, (L,))
    t1 = lax.broadcast(g0 + 1, (L,))
    t2 = lax.broadcast(g0 + 2, (L,))
    zero = jnp.zeros((L,), jnp.int32)
    sh = jnp.full((L,), 31, jnp.int32)

    def cnt(i, carry):
        a0, a1, a2 = carry
        v = bbuf[pl.ds(i * L, L)]
        a0 = a0 - lax.shift_right_arithmetic(v - t0, sh)
        a1 = a1 - lax.shift_right_arithmetic(v - t1, sh)
        a2 = a2 - lax.shift_right_arithmetic(v - t2, sh)
        return a0, a1, a2

    a0, a1, a2 = lax.fori_loop(0, N // L, cnt, (zero, zero, zero))

    def lane_sum(v):
        t = v[0]
        for i in range(1, L):
            t = t + v[i]
        return t

    s0 = lane_sum(a0)
    s1 = lane_sum(a1)
    s2 = lane_sum(a2)

    zf = jnp.zeros((L,), jnp.float32)

    for (start, end, gout) in ((s0, s1, 0), (s1, s2, 1)):
        astart = pl.multiple_of((start // 8) * 8, 8)
        length = end - astart
        nch = (length + GCHUNK - 1) // GCHUNK

        def chunk(k, m, astart=astart, start=start, end=end):
            off = pl.multiple_of(astart + k * GCHUNK, 8)
            pltpu.sync_copy(h_hbm.at[pl.ds(off, GCHUNK)], cbuf)

            def row(r, m):
                grow = off + r
                lo = 1 + lax.shift_right_arithmetic(grow - start, 31)
                hi = 1 + lax.shift_right_arithmetic(end - 1 - grow, 31)
                validf = lax.broadcast((lo * hi).astype(jnp.float32), (L,))
                out = []
                for j in range(D2 // L):
                    v = cbuf[r, pl.ds(j * L, L)]
                    out.append(jnp.maximum(m[j], v * validf))
                return tuple(out)

            return lax.fori_loop(0, GCHUNK, row, m)

        m = lax.fori_loop(0, nch, chunk, tuple(zf for _ in range(D2 // L)))
        for j in range(D2 // L):
            obuf[0, pl.ds(j * L, L)] = m[j]
        pltpu.sync_copy(obuf, out_hbm.at[g0 + gout])


_segmax_kernel = pl.kernel(
    _segmax_body,
    out_type=jax.ShapeDtypeStruct((NG, 1, D2), jnp.float32),
    mesh=_mesh,
    scratch_types=[
        pltpu.VMEM((N,), jnp.int32),
        pltpu.VMEM((GCHUNK, D2), jnp.float32),
        pltpu.VMEM((1, D2), jnp.float32),
    ],
)


def _dinv_from(deg_blk):
    deg = deg_blk[0, :, 0] + deg_blk[1, :, 0] + 1.0
    return lax.rsqrt(deg)


def _tc1_body(deg_ref, x_ref, w_ref, g_ref):
    dinv = _dinv_from(deg_ref[...])
    xs = x_ref[...] * dinv[:, None]
    g_ref[...] = jnp.dot(xs, w_ref[...], preferred_element_type=jnp.float32)


_tc1 = pl.pallas_call(
    _tc1_body,
    grid=(GRID,),
    in_specs=[
        pl.BlockSpec((NC, BR, L), lambda i: (0, i, 0)),
        pl.BlockSpec((BR, D), lambda i: (i, 0)),
        pl.BlockSpec((D, D), lambda i: (0, 0)),
    ],
    out_specs=pl.BlockSpec((BR, D), lambda i: (i, 0)),
    out_shape=jax.ShapeDtypeStruct((NPAD, D), jnp.float32),
)


def _tc2_body(deg_ref, a_ref, g_ref, b_ref, w_ref, o_ref):
    dinv = _dinv_from(deg_ref[...])
    a = a_ref[...]
    h = dinv[:, None] * (a[0] + a[1] + g_ref[...]) + b_ref[...]
    h = jnp.maximum(h, 0.0) * dinv[:, None]
    g2 = jnp.dot(h, w_ref[...], preferred_element_type=jnp.float32)
    o_ref[0] = g2[:, :D]
    o_ref[1] = g2[:, D:]


_tc2 = pl.pallas_call(
    _tc2_body,
    grid=(GRID,),
    in_specs=[
        pl.BlockSpec((NC, BR, L), lambda i: (0, i, 0)),
        pl.BlockSpec((NC, BR, D), lambda i: (0, i, 0)),
        pl.BlockSpec((BR, D), lambda i: (i, 0)),
        pl.BlockSpec((1, D), lambda i: (0, 0)),
        pl.BlockSpec((D, 2 * D), lambda i: (0, 0)),
    ],
    out_specs=pl.BlockSpec((NC, BR, D), lambda i: (0, i, 0)),
    out_shape=jax.ShapeDtypeStruct((NC, NPAD, D), jnp.float32),
)


def _tc3_body(deg_ref, a_ref, g_ref, b_ref, o_ref):
    dinv = _dinv_from(deg_ref[...])
    a = a_ref[...]
    g = g_ref[...]
    b = b_ref[...]
    lo = jnp.maximum(dinv[:, None] * (a[0] + g[0]) + b[:, :D], 0.0)
    hi = jnp.maximum(dinv[:, None] * (a[1] + g[1]) + b[:, D:], 0.0)
    o_ref[...] = jnp.concatenate([lo, hi], axis=1)


_tc3 = pl.pallas_call(
    _tc3_body,
    grid=(GRID,),
    in_specs=[
        pl.BlockSpec((NC, BR, L), lambda i: (0, i, 0)),
        pl.BlockSpec((NC, BR, D), lambda i: (0, i, 0)),
        pl.BlockSpec((NC, BR, D), lambda i: (0, i, 0)),
        pl.BlockSpec((1, 2 * D), lambda i: (0, 0)),
    ],
    out_specs=pl.BlockSpec((BR, 2 * D), lambda i: (i, 0)),
    out_shape=jax.ShapeDtypeStruct((NPAD, 2 * D), jnp.float32),
)


def _head_body(gm_ref, wg1_ref, bg1_ref, wg2_ref, bg2_ref, wo_ref, bo_ref, o_ref):
    z = jnp.dot(gm_ref[...], wg1_ref[...], preferred_element_type=jnp.float32)
    z = jnp.maximum(z + bg1_ref[...], 0.0)
    z = jnp.dot(z, wg2_ref[...], preferred_element_type=jnp.float32) + bg2_ref[...]
    o_ref[...] = jnp.dot(z, wo_ref[...], preferred_element_type=jnp.float32) + bo_ref[...]


_head = pl.pallas_call(
    _head_body,
    out_shape=jax.ShapeDtypeStruct((NG, 1), jnp.float32),
)


@jax.jit
def kernel(x, edge_index, batch, W1, b1, W2, b2, Wg1, bg1, Wg2, bg2, Wo, bo):
    src = edge_index[0]
    dst = edge_index[1]
    xpad = jnp.zeros((NPAD, D), x.dtype).at[:N].set(x)

    deg = _deg_kernel(dst)
    g1 = _tc1(deg, xpad, W1)
    a1 = _agg1_kernel(src, dst, g1)
    g2 = _tc2(deg, a1, g1, b1.reshape(1, D), W2)
    a2 = _agg2_kernel(src, dst, g2[0], g2[1])
    h2 = _tc3(deg, a2, g2, b2.reshape(1, 2 * D))
    gm = _segmax_kernel(batch, h2)
    out = _head(gm.reshape(NG, D2), Wg1, bg1.reshape(1, 1024), Wg2,
                bg2.reshape(1, D), Wo, bo.reshape(1, 1))
    return out


# speedup vs baseline: 14.4727x; 1.1709x over previous
import jax
import jax.numpy as jnp
from jax import lax
from jax.experimental import pallas as pl
from jax.experimental.pallas import tpu as pltpu
from jax.experimental.pallas import tpu_sc as plsc

N = 10000
E = 320000
D = 128
NG = 64

NC = 2
NS = 16
L = 16
NW = NC * NS

CH = 128

BR = 512
GRID = 20
NPAD = GRID * BR
RPT = NPAD // NS

_mesh = plsc.VectorSubcoreMesh(
    core_axis_name="c", subcore_axis_name="s", num_cores=NC, num_subcores=NS)


def _zero_vmem_2d(ref, nrows, ncols):
    npc = ncols // L

    def body(i, _):
        ref[i // npc, pl.ds((i % npc) * L, L)] = jnp.zeros((L,), jnp.float32)
        return 0

    lax.fori_loop(0, nrows * npc, body, 0)


def _fill_spmem_zero(acc, zbuf, row0, nrows, zrows):
    nfull = nrows // zrows
    rem = nrows - nfull * zrows
    for j in range(nfull):
        pltpu.sync_copy(zbuf.at[pl.ds(0, zrows)],
                        acc.at[pl.ds(row0 + j * zrows, zrows)])
    if rem:
        pltpu.sync_copy(zbuf.at[pl.ds(0, rem)],
                        acc.at[pl.ds(row0 + nfull * zrows, rem)])


TOTAL_CHUNKS = E // CH
CPW = TOTAL_CHUNKS // NW
XW = TOTAL_CHUNKS - CPW * NW
CPS = TOTAL_CHUNKS // NS
XS = TOTAL_CHUNKS - CPS * NS


def _chunk_range(idx, per, extra):
    start = idx * per + jnp.minimum(idx, extra)
    n = per - lax.shift_right_arithmetic(idx - extra, 31)
    return start, n


def _deg_body(dst_hbm, out_hbm, ones_v, zbuf, dbuf0, dbuf1, ss0, ss1, acc):
    c = lax.axis_index("c")
    s = lax.axis_index("s")
    wid = c * NS + s
    cstart, ncnk = _chunk_range(wid, CPW, XW)
    row0 = s * RPT

    _zero_vmem_2d(zbuf, CH, L)
    _fill_spmem_zero(acc, zbuf, row0, RPT, CH)

    def fill1(i, _):
        ones_v[i] = jnp.ones((L,), jnp.float32)
        return 0
    lax.fori_loop(0, CH, fill1, 0)

    def zidx(i, _):
        dbuf0[pl.ds(i * L, L)] = jnp.zeros((L,), jnp.int32)
        dbuf1[pl.ds(i * L, L)] = jnp.zeros((L,), jnp.int32)
        return 0
    lax.fori_loop(0, CH // L, zidx, 0)

    plsc.subcore_barrier()

    dbufs = (dbuf0, dbuf1)
    sss = (ss0, ss1)
    for b in range(2):
        pltpu.make_async_copy(zbuf, acc.at[dbufs[b]], sss[b]).start(add=True)

    def do_chunk(k, b):
        off = (cstart + k) * CH
        pltpu.make_async_copy(zbuf, acc.at[dbufs[b]], sss[b]).wait()
        pltpu.sync_copy(dst_hbm.at[pl.ds(off, CH)], dbufs[b])
        pltpu.make_async_copy(ones_v, acc.at[dbufs[b]], sss[b]).start(add=True)

    npairs = ncnk // 2

    def pair(k2, _):
        do_chunk(k2 * 2, 0)
        do_chunk(k2 * 2 + 1, 1)
        return 0
    lax.fori_loop(0, npairs, pair, 0)

    @pl.when(ncnk != npairs * 2)
    def _():
        do_chunk(npairs * 2, 0)

    for b in range(2):
        pltpu.make_async_copy(zbuf, acc.at[dbufs[b]], sss[b]).wait()

    plsc.subcore_barrier()
    pltpu.sync_copy(acc.at[pl.ds(row0, RPT)],
                    out_hbm.at[c, pl.ds(row0, RPT)])


_deg_kernel = pl.kernel(
    _deg_body,
    out_type=jax.ShapeDtypeStruct((NC, NPAD, L), jnp.float32),
    mesh=_mesh,
    scratch_types=[
        pltpu.VMEM((CH, L), jnp.float32),
        pltpu.VMEM((CH, L), jnp.float32),
        pltpu.VMEM((CH,), jnp.int32),
        pltpu.VMEM((CH,), jnp.int32),
        pltpu.SemaphoreType.DMA,
        pltpu.SemaphoreType.DMA,
        pltpu.VMEM_SHARED((NPAD, L), jnp.float32),
    ],
)


def _pipe_gather_scatter(src_hbm, dst_hbm, g_hbm, acc,
                         sbufs, dbufs, rowss, sgs, sss, cstart, ncnk):
    for b in range(2):
        pltpu.make_async_copy(rowss[b], acc.at[dbufs[b]], sss[b]).start(add=True)

    def do_chunk(k, b):
        off = (cstart + k) * CH
        pltpu.make_async_copy(rowss[b], acc.at[dbufs[b]], sss[b]).wait()
        pltpu.sync_copy(src_hbm.at[pl.ds(off, CH)], sbufs[b])
        pltpu.sync_copy(dst_hbm.at[pl.ds(off, CH)], dbufs[b])
        g = pltpu.make_async_copy(g_hbm.at[sbufs[b]], rowss[b], sgs[b])
        g.start()
        g.wait()
        pltpu.make_async_copy(rowss[b], acc.at[dbufs[b]], sss[b]).start(add=True)

    npairs = ncnk // 2

    def pair(k2, _):
        do_chunk(k2 * 2, 0)
        do_chunk(k2 * 2 + 1, 1)
        return 0
    lax.fori_loop(0, npairs, pair, 0)

    @pl.when(ncnk != npairs * 2)
    def _():
        do_chunk(npairs * 2, 0)

    for b in range(2):
        pltpu.make_async_copy(rowss[b], acc.at[dbufs[b]], sss[b]).wait()


def _zero_pipe_bufs(sbufs, dbufs, rowss):
    _zero_vmem_2d(rowss[0], CH, D)
    _zero_vmem_2d(rowss[1], CH, D)

    def zidx(i, _):
        for buf in (*sbufs, *dbufs):
            buf[pl.ds(i * L, L)] = jnp.zeros((L,), jnp.int32)
        return 0
    lax.fori_loop(0, CH // L, zidx, 0)


def _agg1_body(src_hbm, dst_hbm, g_hbm, out_hbm,
               sbuf0, sbuf1, dbuf0, dbuf1, rows0, rows1,
               sg0, sg1, ss0, ss1, acc):
    c = lax.axis_index("c")
    s = lax.axis_index("s")
    wid = c * NS + s
    cstart, ncnk = _chunk_range(wid, CPW, XW)
    row0 = s * RPT

    _zero_pipe_bufs((sbuf0, sbuf1), (dbuf0, dbuf1), (rows0, rows1))
    _fill_spmem_zero(acc, rows0, row0, RPT, CH)
    plsc.subcore_barrier()

    _pipe_gather_scatter(src_hbm, dst_hbm, g_hbm, acc,
                         (sbuf0, sbuf1), (dbuf0, dbuf1), (rows0, rows1),
                         (sg0, sg1), (ss0, ss1), cstart, ncnk)

    plsc.subcore_barrier()
    pltpu.sync_copy(acc.at[pl.ds(row0, RPT)],
                    out_hbm.at[c, pl.ds(row0, RPT)])


_AGG_SCRATCH = [
    pltpu.VMEM((CH,), jnp.int32),
    pltpu.VMEM((CH,), jnp.int32),
    pltpu.VMEM((CH,), jnp.int32),
    pltpu.VMEM((CH,), jnp.int32),
    pltpu.VMEM((CH, D), jnp.float32),
    pltpu.VMEM((CH, D), jnp.float32),
    pltpu.SemaphoreType.DMA,
    pltpu.SemaphoreType.DMA,
    pltpu.SemaphoreType.DMA,
    pltpu.SemaphoreType.DMA,
    pltpu.VMEM_SHARED((NPAD, D), jnp.float32),
]

_agg1_kernel = pl.kernel(
    _agg1_body,
    out_type=jax.ShapeDtypeStruct((NC, NPAD, D), jnp.float32),
    mesh=_mesh,
    scratch_types=list(_AGG_SCRATCH),
)


def _agg2_body(src_hbm, dst_hbm, ga_hbm, gb_hbm, out_hbm,
               sbuf0, sbuf1, dbuf0, dbuf1, rows0, rows1,
               sg0, sg1, ss0, ss1, acc):
    c = lax.axis_index("c")
    s = lax.axis_index("s")
    cstart, ncnk = _chunk_range(s, CPS, XS)
    row0 = s * RPT

    _zero_pipe_bufs((sbuf0, sbuf1), (dbuf0, dbuf1), (rows0, rows1))
    _fill_spmem_zero(acc, rows0, row0, RPT, CH)
    plsc.subcore_barrier()

    @pl.when(c == 0)
    def _():
        _pipe_gather_scatter(src_hbm, dst_hbm, ga_hbm, acc,
                             (sbuf0, sbuf1), (dbuf0, dbuf1), (rows0, rows1),
                             (sg0, sg1), (ss0, ss1), cstart, ncnk)

    @pl.when(c == 1)
    def _():
        _pipe_gather_scatter(src_hbm, dst_hbm, gb_hbm, acc,
                             (sbuf0, sbuf1), (dbuf0, dbuf1), (rows0, rows1),
                             (sg0, sg1), (ss0, ss1), cstart, ncnk)

    plsc.subcore_barrier()
    pltpu.sync_copy(acc.at[pl.ds(row0, RPT)],
                    out_hbm.at[c, pl.ds(row0, RPT)])


_agg2_kernel = pl.kernel(
    _agg2_body,
    out_type=jax.ShapeDtypeStruct((NC, NPAD, D), jnp.float32),
    mesh=_mesh,
    scratch_types=list(_AGG_SCRATCH),
)


GCHUNK = 64
D2 = 256


def _segmax_body(batch_hbm, h_hbm, out_hbm, bbuf, cbuf, obuf):
    wid = lax.axis_index("c") * NS + lax.axis_index("s")
    g0 = wid * 2

    pltpu.sync_copy(batch_hbm, bbuf)

    t0 = lax.broadcast(g0, (L,))
    t1 = lax.broadcast(g0 + 1, (L,))
    t2 = lax.broadcast(g0 + 2, (L,))
    zero = jnp.zeros((L,), jnp.int32)
    sh = jnp.full((L,), 31, jnp.int32)

    def cnt(i, carry):
        a0, a1, a2 = carry
        v = bbuf[pl.ds(i * L, L)]
        a0 = a0 - lax.shift_right_arithmetic(v - t0, sh)
        a1 = a1 - lax.shift_right_arithmetic(v - t1, sh)
        a2 = a2 - lax.shift_right_arithmetic(v - t2, sh)
        return a0, a1, a2

    a0, a1, a2 = lax.fori_loop(0, N // L, cnt, (zero, zero, zero))

    def lane_sum(v):
        t = v[0]
        for i in range(1, L):
            t = t + v[i]
        return t

    s0 = lane_sum(a0)
    s1 = lane_sum(a1)
    s2 = lane_sum(a2)

    zf = jnp.zeros((L,), jnp.float32)

    for (start, end, gout) in ((s0, s1, 0), (s1, s2, 1)):
        astart = pl.multiple_of((start // 8) * 8, 8)
        length = end - astart
        nch = (length + GCHUNK - 1) // GCHUNK

        def chunk(k, m, astart=astart, start=start, end=end):
            off = pl.multiple_of(astart + k * GCHUNK, 8)
            pltpu.sync_copy(h_hbm.at[pl.ds(off, GCHUNK)], cbuf)

            def row(r, m):
                grow = off + r
                lo = 1 + lax.shift_right_arithmetic(grow - start, 31)
                hi = 1 + lax.shift_right_arithmetic(end - 1 - grow, 31)
                validf = lax.broadcast((lo * hi).astype(jnp.float32), (L,))
                out = []
                for j in range(D2 // L):
                    v = cbuf[r, pl.ds(j * L, L)]
                    out.append(jnp.maximum(m[j], v * validf))
                return tuple(out)

            return lax.fori_loop(0, GCHUNK, row, m)

        m = lax.fori_loop(0, nch, chunk, tuple(zf for _ in range(D2 // L)))
        for j in range(D2 // L):
            obuf[0, pl.ds(j * L, L)] = m[j]
        pltpu.sync_copy(obuf, out_hbm.at[g0 + gout])


_segmax_kernel = pl.kernel(
    _segmax_body,
    out_type=jax.ShapeDtypeStruct((NG, 1, D2), jnp.float32),
    mesh=_mesh,
    scratch_types=[
        pltpu.VMEM((N,), jnp.int32),
        pltpu.VMEM((GCHUNK, D2), jnp.float32),
        pltpu.VMEM((1, D2), jnp.float32),
    ],
)


def _dinv_from(deg_blk):
    deg = deg_blk[0, :, 0] + deg_blk[1, :, 0] + 1.0
    return lax.rsqrt(deg)


def _tc1_body(deg_ref, x_ref, w_ref, g_ref):
    dinv = _dinv_from(deg_ref[...])
    xs = x_ref[...] * dinv[:, None]
    g_ref[...] = jnp.dot(xs, w_ref[...], preferred_element_type=jnp.float32)


_tc1 = pl.pallas_call(
    _tc1_body,
    grid=(GRID,),
    in_specs=[
        pl.BlockSpec((NC, BR, L), lambda i: (0, i, 0)),
        pl.BlockSpec((BR, D), lambda i: (i, 0)),
        pl.BlockSpec((D, D), lambda i: (0, 0)),
    ],
    out_specs=pl.BlockSpec((BR, D), lambda i: (i, 0)),
    out_shape=jax.ShapeDtypeStruct((NPAD, D), jnp.float32),
)


def _tc2_body(deg_ref, a_ref, g_ref, b_ref, w_ref, o_ref):
    dinv = _dinv_from(deg_ref[...])
    a = a_ref[...]
    h = dinv[:, None] * (a[0] + a[1] + g_ref[...]) + b_ref[...]
    h = jnp.maximum(h, 0.0) * dinv[:, None]
    g2 = jnp.dot(h, w_ref[...], preferred_element_type=jnp.float32)
    o_ref[0] = g2[:, :D]
    o_ref[1] = g2[:, D:]


_tc2 = pl.pallas_call(
    _tc2_body,
    grid=(GRID,),
    in_specs=[
        pl.BlockSpec((NC, BR, L), lambda i: (0, i, 0)),
        pl.BlockSpec((NC, BR, D), lambda i: (0, i, 0)),
        pl.BlockSpec((BR, D), lambda i: (i, 0)),
        pl.BlockSpec((1, D), lambda i: (0, 0)),
        pl.BlockSpec((D, 2 * D), lambda i: (0, 0)),
    ],
    out_specs=pl.BlockSpec((NC, BR, D), lambda i: (0, i, 0)),
    out_shape=jax.ShapeDtypeStruct((NC, NPAD, D), jnp.float32),
)


def _tc3_body(deg_ref, a_ref, g_ref, b_ref, o_ref):
    dinv = _dinv_from(deg_ref[...])
    a = a_ref[...]
    g = g_ref[...]
    b = b_ref[...]
    lo = jnp.maximum(dinv[:, None] * (a[0] + g[0]) + b[:, :D], 0.0)
    hi = jnp.maximum(dinv[:, None] * (a[1] + g[1]) + b[:, D:], 0.0)
    o_ref[...] = jnp.concatenate([lo, hi], axis=1)


_tc3 = pl.pallas_call(
    _tc3_body,
    grid=(GRID,),
    in_specs=[
        pl.BlockSpec((NC, BR, L), lambda i: (0, i, 0)),
        pl.BlockSpec((NC, BR, D), lambda i: (0, i, 0)),
        pl.BlockSpec((NC, BR, D), lambda i: (0, i, 0)),
        pl.BlockSpec((1, 2 * D), lambda i: (0, 0)),
    ],
    out_specs=pl.BlockSpec((BR, 2 * D), lambda i: (i, 0)),
    out_shape=jax.ShapeDtypeStruct((NPAD, 2 * D), jnp.float32),
)


def _head_body(gm_ref, wg1_ref, bg1_ref, wg2_ref, bg2_ref, wo_ref, bo_ref, o_ref):
    z = jnp.dot(gm_ref[...], wg1_ref[...], preferred_element_type=jnp.float32)
    z = jnp.maximum(z + bg1_ref[...], 0.0)
    z = jnp.dot(z, wg2_ref[...], preferred_element_type=jnp.float32) + bg2_ref[...]
    o_ref[...] = jnp.dot(z, wo_ref[...], preferred_element_type=jnp.float32) + bo_ref[...]


_head = pl.pallas_call(
    _head_body,
    out_shape=jax.ShapeDtypeStruct((NG, 1), jnp.float32),
)


@jax.jit
def kernel(x, edge_index, batch, W1, b1, W2, b2, Wg1, bg1, Wg2, bg2, Wo, bo):
    src = edge_index[0]
    dst = edge_index[1]
    xpad = jnp.zeros((NPAD, D), x.dtype).at[:N].set(x)

    deg = _deg_kernel(dst)
    g1 = _tc1(deg, xpad, W1)
    a1 = _agg1_kernel(src, dst, g1)
    g2 = _tc2(deg, a1, g1, b1.reshape(1, D), W2)
    a2 = _agg2_kernel(src, dst, g2[0], g2[1])
    h2 = _tc3(deg, a2, g2, b2.reshape(1, 2 * D))
    gm = _segmax_kernel(batch, h2)
    out = _head(gm.reshape(NG, D2), Wg1, bg1.reshape(1, 1024), Wg2,
                bg2.reshape(1, D), Wo, bo.reshape(1, 1))
    return out
